# baseline (speedup 1.0000x reference)
import functools

import jax
import jax.numpy as jnp
from jax import lax
from jax.experimental import pallas as pl
from jax.experimental.pallas import tpu as pltpu

_START = 0.5


def _threefry2x32(x0, x1):
    k0 = jnp.uint32(0)
    k1 = jnp.uint32(0)
    k2 = jnp.uint32(0x1BD11BDA)
    ks = (k0, k1, k2)
    rotations = ((13, 15, 26, 6), (17, 29, 16, 24))
    x0 = x0 + ks[0]
    x1 = x1 + ks[1]
    for i in range(5):
        for r in rotations[i % 2]:
            x0 = x0 + x1
            x1 = (x1 << r) | (x1 >> (32 - r))
            x1 = x1 ^ x0
        x0 = x0 + ks[(i + 1) % 3]
        x1 = x1 + ks[(i + 2) % 3] + jnp.uint32(i + 1)
    return x0, x1


def _permute_kernel(x_ref, o_ref, *, T, W, HF, total):
    b = pl.program_id(0)
    tb = pl.program_id(1)

    xb = x_ref[0]
    o_ref[0, :, :HF] = xb[:, :HF]

    val = xb[:, HF:].T

    base = (b * T + tb * W) * HF
    ii = lax.broadcasted_iota(jnp.int32, (HF, W), 0)
    tt = lax.broadcasted_iota(jnp.int32, (HF, W), 1)
    cnt = (base + tt * HF + ii).astype(jnp.uint32)

    o0, o1 = _threefry2x32(jnp.zeros_like(cnt), cnt)
    bits = o0 ^ o1

    key = (((bits >> 9) << 8) | ii.astype(jnp.uint32)).astype(jnp.int32)

    N = HF
    bit = [(ii & (1 << l)) != 0 for l in range(8)]
    k = 2
    while k <= N:
        j = k // 2
        while j >= 1:
            lj = j.bit_length() - 1
            kp = pltpu.roll(key, N - j, 0)
            km = pltpu.roll(key, j, 0)
            vp = pltpu.roll(val, N - j, 0)
            vm = pltpu.roll(val, j, 0)
            upper = bit[lj]
            kprt = jnp.where(upper, km, kp)
            vprt = jnp.where(upper, vm, vp)
            gt = key > kprt
            if k == N:
                cmask = upper
            else:
                cmask = jnp.logical_xor(upper, bit[k.bit_length() - 1])
            take = jnp.logical_xor(gt, cmask)
            key = jnp.where(take, kprt, key)
            val = jnp.where(take, vprt, val)
            j //= 2
        k *= 2

    o_ref[0, :, HF:] = val.T


@jax.jit
def kernel(x):
    B, T, F = x.shape
    start_bin = int(_START * F)
    HF = F - start_bin
    W = min(128, T)
    total = B * T * HF
    kfn = functools.partial(_permute_kernel, T=T, W=W, HF=HF, total=total)
    return pl.pallas_call(
        kfn,
        grid=(B, T // W),
        in_specs=[pl.BlockSpec((1, W, F), lambda b, t: (b, t, 0))],
        out_specs=pl.BlockSpec((1, W, F), lambda b, t: (b, t, 0)),
        out_shape=jax.ShapeDtypeStruct((B, T, F), x.dtype),
    )(x)


# speedup vs baseline: 1.0455x; 1.0455x over previous
import functools

import jax
import jax.numpy as jnp
from jax import lax
from jax.experimental import pallas as pl
from jax.experimental.pallas import tpu as pltpu

_START = 0.5


def _threefry2x32(x0, x1):
    k0 = jnp.uint32(0)
    k1 = jnp.uint32(0)
    k2 = jnp.uint32(0x1BD11BDA)
    ks = (k0, k1, k2)
    rotations = ((13, 15, 26, 6), (17, 29, 16, 24))
    x0 = x0 + ks[0]
    x1 = x1 + ks[1]
    for i in range(5):
        for r in rotations[i % 2]:
            x0 = x0 + x1
            x1 = (x1 << r) | (x1 >> (32 - r))
            x1 = x1 ^ x0
        x0 = x0 + ks[(i + 1) % 3]
        x1 = x1 + ks[(i + 2) % 3] + jnp.uint32(i + 1)
    return x0, x1


def _permute_kernel(x_ref, o_ref, *, T, W, HF, total):
    b = pl.program_id(0)
    tb = pl.program_id(1)

    xb = x_ref[0]
    o_ref[0, :, :HF] = xb[:, :HF]

    val = xb[:, HF:].T

    base = (b * T + tb * W) * HF
    ii = lax.broadcasted_iota(jnp.int32, (HF, W), 0)
    tt = lax.broadcasted_iota(jnp.int32, (HF, W), 1)
    cnt = (base + tt * HF + ii).astype(jnp.uint32)

    o0, o1 = _threefry2x32(jnp.zeros_like(cnt), cnt)
    bits = o0 ^ o1

    key = (((bits >> 9) << 8) | ii.astype(jnp.uint32)).astype(jnp.int32)

    N = HF
    bit = [(ii & (1 << l)) != 0 for l in range(8)]
    k = 2
    while k <= N:
        j = k // 2
        while j >= 1:
            lj = j.bit_length() - 1
            upper = bit[lj]
            kp = pltpu.roll(key, N - j, 0)
            km = pltpu.roll(key, j, 0)
            kprt = jnp.where(upper, km, kp)
            vp = pltpu.roll(val, N - j, 0)
            vm = pltpu.roll(val, j, 0)
            vprt = jnp.where(upper, vm, vp)
            gt = key > kprt
            if k == N:
                cmask = upper
            else:
                cmask = jnp.logical_xor(upper, bit[k.bit_length() - 1])
            take = jnp.logical_xor(gt, cmask)
            key = jnp.where(take, kprt, key)
            val = jnp.where(take, vprt, val)
            j //= 2
        k *= 2

    o_ref[0, :, HF:] = val.T


@jax.jit
def kernel(x):
    B, T, F = x.shape
    start_bin = int(_START * F)
    HF = F - start_bin
    W = min(512, T)
    total = B * T * HF
    kfn = functools.partial(_permute_kernel, T=T, W=W, HF=HF, total=total)
    return pl.pallas_call(
        kfn,
        grid=(B, T // W),
        in_specs=[pl.BlockSpec((1, W, F), lambda b, t: (b, t, 0))],
        out_specs=pl.BlockSpec((1, W, F), lambda b, t: (b, t, 0)),
        out_shape=jax.ShapeDtypeStruct((B, T, F), x.dtype),
    )(x)


# speedup vs baseline: 1.0479x; 1.0023x over previous
import functools

import jax
import jax.numpy as jnp
from jax import lax
from jax.experimental import pallas as pl
from jax.experimental.pallas import tpu as pltpu

_START = 0.5


def _threefry2x32(x0, x1):
    k0 = jnp.uint32(0)
    k1 = jnp.uint32(0)
    k2 = jnp.uint32(0x1BD11BDA)
    ks = (k0, k1, k2)
    rotations = ((13, 15, 26, 6), (17, 29, 16, 24))
    x0 = x0 + ks[0]
    x1 = x1 + ks[1]
    for i in range(5):
        for r in rotations[i % 2]:
            x0 = x0 + x1
            x1 = (x1 << r) | (x1 >> (32 - r))
            x1 = x1 ^ x0
        x0 = x0 + ks[(i + 1) % 3]
        x1 = x1 + ks[(i + 2) % 3] + jnp.uint32(i + 1)
    return x0, x1


def _permute_kernel(x_ref, o_ref, *, T, W, HF, total):
    b = pl.program_id(0)
    tb = pl.program_id(1)

    xb = x_ref[0]
    o_ref[0, :, :HF] = xb[:, :HF]

    val = xb[:, HF:].T

    base = (b * T + tb * W) * HF
    ii = lax.broadcasted_iota(jnp.int32, (HF, W), 0)
    tt = lax.broadcasted_iota(jnp.int32, (HF, W), 1)
    cnt = (base + tt * HF + ii).astype(jnp.uint32)

    o0, o1 = _threefry2x32(jnp.zeros_like(cnt), cnt)
    bits = o0 ^ o1

    key = (((bits >> 9) << 8) | ii.astype(jnp.uint32)).astype(jnp.int32)

    N = HF
    bit = [(ii & (1 << l)) != 0 for l in range(8)]
    k = 2
    while k <= N:
        j = k // 2
        while j >= 1:
            lj = j.bit_length() - 1
            upper = bit[lj]
            kp = pltpu.roll(key, N - j, 0)
            km = pltpu.roll(key, j, 0)
            kprt = jnp.where(upper, km, kp)
            vp = pltpu.roll(val, N - j, 0)
            vm = pltpu.roll(val, j, 0)
            vprt = jnp.where(upper, vm, vp)
            gt = key > kprt
            if k == N:
                cmask = upper
            else:
                cmask = jnp.logical_xor(upper, bit[k.bit_length() - 1])
            take = jnp.logical_xor(gt, cmask)
            key = jnp.where(take, kprt, key)
            val = jnp.where(take, vprt, val)
            j //= 2
        k *= 2

    o_ref[0, :, HF:] = val.T


@jax.jit
def kernel(x):
    B, T, F = x.shape
    start_bin = int(_START * F)
    HF = F - start_bin
    W = min(1024, T)
    total = B * T * HF
    kfn = functools.partial(_permute_kernel, T=T, W=W, HF=HF, total=total)
    return pl.pallas_call(
        kfn,
        grid=(B, T // W),
        in_specs=[pl.BlockSpec((1, W, F), lambda b, t: (b, t, 0))],
        out_specs=pl.BlockSpec((1, W, F), lambda b, t: (b, t, 0)),
        out_shape=jax.ShapeDtypeStruct((B, T, F), x.dtype),
    )(x)
